# baseline (speedup 1.0000x reference)
import jax
import jax.numpy as jnp
from jax import lax
from jax.experimental import pallas as pl
from jax.experimental.pallas import tpu as pltpu
from jax.experimental.pallas import tpu_sc as plsc

_NC = 2
_NS = 16
_K = 128
_L = 16
_W = 8


def _chunks_per_worker(e):
    per_round = _NS * _K * 2 * _W
    return (-(-e // per_round)) * 2 * _W


def _make_segsum(n_rows, n_pad, f, cpw, with_deg):
    stripe = n_pad // _NS
    nw1 = cpw // _W
    mesh = plsc.VectorSubcoreMesh(core_axis_name="core", subcore_axis_name="subcore")
    out_type = [jax.ShapeDtypeStruct((_NC, n_pad, f), jnp.float32)]
    scratch = [
        pltpu.VMEM_SHARED((n_pad, f), jnp.float32),
        pltpu.VMEM((2 * _W, _K), jnp.int32),
        pltpu.VMEM((2 * _W, _K), jnp.int32),
        pltpu.VMEM((_K, f), jnp.float32),
        pltpu.VMEM((_K, f), jnp.float32),
        pltpu.SemaphoreType.DMA,
        pltpu.SemaphoreType.DMA,
        pltpu.SemaphoreType.DMA,
        pltpu.SemaphoreType.DMA,
    ]
    if with_deg:
        out_type.append(jax.ShapeDtypeStruct((_NC, n_pad), jnp.float32))
        scratch += [
            pltpu.VMEM((_K,), jnp.float32),
            pltpu.VMEM((stripe,), jnp.float32),
            pltpu.VMEM_SHARED((n_pad,), jnp.float32),
        ]

    def body(vals, adj1, out, *rest):
        if with_deg:
            deg_out = rest[0]
            rest = rest[1:]
        acc, ib0, ib1, r0, r1, g0, g1, i0, i1 = rest[:9]
        if with_deg:
            ones, zdeg, accd = rest[9:]
        idxb = (ib0, ib1)
        rows = (r0, r1)
        gsem = (g0, g1)
        isem = (i0, i1)
        c = lax.axis_index("core")
        s = lax.axis_index("subcore")

        zvec = jnp.zeros((_L,), jnp.float32)

        @pl.loop(0, 64)
        def _(i):
            @pl.loop(0, f, step=_L)
            def _(j):
                r0[i, pl.ds(j, _L)] = zvec

        @pl.loop(0, stripe, step=64)
        def _(r):
            pltpu.sync_copy(r0.at[pl.ds(0, 64)],
                            acc.at[pl.ds(s * stripe + r, 64)])

        if with_deg:
            ovec = jnp.ones((_L,), jnp.float32)

            @pl.loop(0, _K, step=_L)
            def _(j):
                ones[pl.ds(j, _L)] = ovec

            @pl.loop(0, stripe, step=_L)
            def _(j):
                zdeg[pl.ds(j, _L)] = zvec

            pltpu.sync_copy(zdeg, accd.at[pl.ds(s * stripe, stripe)])

        plsc.subcore_barrier()

        @pl.when(c == 1)
        def _():
            pltpu.sync_copy(adj1.at[s, 0], idxb[0])
            pltpu.async_copy(adj1.at[s, 1], idxb[1], isem[1])
            pltpu.async_copy(vals.at[idxb[0].at[0]], rows[0], gsem[0])
            pltpu.async_copy(vals.at[idxb[0].at[1]], rows[1], gsem[1])

            def half(w, p):
                ib = idxb[p]
                ibn = idxb[1 - p]
                for k in range(_W):
                    b = k % 2
                    pltpu.make_async_copy(
                        vals.at[ib.at[k]], rows[b], gsem[b]).wait()
                    pltpu.sync_copy(rows[b], acc.at[ib.at[_W + k]], add=True)
                    if with_deg:
                        pltpu.sync_copy(ones, accd.at[ib.at[_W + k]],
                                        add=True)
                    if k + 2 < _W:
                        pltpu.async_copy(vals.at[ib.at[k + 2]], rows[b],
                                         gsem[b])
                    elif k == _W - 2:
                        @pl.when(w + 1 < nw1)
                        def _():
                            pltpu.make_async_copy(adj1.at[s, w + 1], ibn,
                                                  isem[1 - p]).wait()
                            pltpu.async_copy(vals.at[ibn.at[0]], rows[b],
                                             gsem[b])
                    else:
                        @pl.when(w + 1 < nw1)
                        def _():
                            pltpu.async_copy(vals.at[ibn.at[1]], rows[b],
                                             gsem[b])

                @pl.when(w + 2 < nw1)
                def _():
                    pltpu.async_copy(adj1.at[s, w + 2], idxb[p], isem[p])

            @pl.loop(0, nw1, step=2)
            def _(w):
                half(w, 0)
                half(w + 1, 1)

        plsc.subcore_barrier()

        pltpu.sync_copy(acc.at[pl.ds(s * stripe, stripe)],
                        out.at[c, pl.ds(s * stripe, stripe)])
        if with_deg:
            pltpu.sync_copy(accd.at[pl.ds(s * stripe, stripe)],
                            deg_out.at[c, pl.ds(s * stripe, stripe)])

    return pl.kernel(body, out_type=tuple(out_type), mesh=mesh,
                     scratch_types=scratch)


def _tc_layer1(s1, deg3, W1, b1, blk):
    n_pad, f_in = s1.shape[1], s1.shape[2]
    h_dim = W1.shape[1]

    def body(p_ref, d_ref, w1_ref, b1_ref, o_ref):
        ssum = p_ref[0] + p_ref[1]
        dsum = d_ref[0] + d_ref[1]
        norm = 1.0 / jnp.maximum(dsum, 1.0)
        agg = jnp.dot(ssum, w1_ref[...], preferred_element_type=jnp.float32)
        agg = (agg + dsum * b1_ref[...]) * norm
        o_ref[...] = jnp.maximum(agg, 0.0)

    return pl.pallas_call(
        body,
        grid=(n_pad // blk,),
        in_specs=[
            pl.BlockSpec((_NC, blk, f_in), lambda i: (0, i, 0)),
            pl.BlockSpec((_NC, blk, 1), lambda i: (0, i, 0)),
            pl.BlockSpec((f_in, h_dim), lambda i: (0, 0)),
            pl.BlockSpec((1, h_dim), lambda i: (0, 0)),
        ],
        out_specs=pl.BlockSpec((blk, h_dim), lambda i: (i, 0)),
        out_shape=jax.ShapeDtypeStruct((n_pad, h_dim), jnp.float32),
    )(s1, deg3, W1, b1.reshape(1, h_dim))


def _tc_layer2(s2, deg3, W2, b2, blk):
    n_pad, h_dim = s2.shape[1], s2.shape[2]
    c_dim = W2.shape[1]

    def body(p_ref, d_ref, w2_ref, b2_ref, o_ref):
        ssum = p_ref[0] + p_ref[1]
        dsum = d_ref[0] + d_ref[1]
        norm = 1.0 / jnp.maximum(dsum, 1.0)
        agg = jnp.dot(ssum, w2_ref[...], preferred_element_type=jnp.float32)
        v = (agg + dsum * b2_ref[...]) * norm
        m = jnp.max(v, axis=1, keepdims=True)
        e = jnp.exp(v - m)
        lse = jnp.log(jnp.sum(e, axis=1, keepdims=True))
        o_ref[...] = (v - m) - lse

    return pl.pallas_call(
        body,
        grid=(n_pad // blk,),
        in_specs=[
            pl.BlockSpec((_NC, blk, h_dim), lambda i: (0, i, 0)),
            pl.BlockSpec((_NC, blk, 1), lambda i: (0, i, 0)),
            pl.BlockSpec((h_dim, c_dim), lambda i: (0, 0)),
            pl.BlockSpec((1, c_dim), lambda i: (0, 0)),
        ],
        out_specs=pl.BlockSpec((blk, c_dim), lambda i: (i, 0)),
        out_shape=jax.ShapeDtypeStruct((n_pad, c_dim), jnp.float32),
    )(s2, deg3, W2, b2.reshape(1, c_dim))


def kernel(x, adj, W1, b1, W2, b2):
    n, f_in = x.shape
    h_dim = W1.shape[1]
    c_dim = W2.shape[1]
    e = adj.shape[1]

    n_pad = ((n + 1023) // 1024) * 1024
    cpw = _chunks_per_worker(e)
    e_pad = _NS * cpw * _K

    src = adj[0].astype(jnp.int32)
    dst = adj[1].astype(jnp.int32)
    pad_dst = n + jnp.arange(e_pad - e, dtype=jnp.int32) % (n_pad - n)
    srcf = jnp.concatenate([src, jnp.zeros((e_pad - e,), jnp.int32)])
    dstf = jnp.concatenate([dst, pad_dst])

    nw1 = cpw // _W
    adj1 = jnp.concatenate(
        [srcf.reshape(_NS, nw1, _W, _K),
         dstf.reshape(_NS, nw1, _W, _K)],
        axis=2)

    seg1 = _make_segsum(n, n_pad, f_in, cpw, with_deg=True)
    s1, deg = seg1(x, adj1)
    deg3 = deg.reshape(_NC, n_pad, 1)

    h = _tc_layer1(s1, deg3, W1, b1, blk=512)

    seg2 = _make_segsum(n_pad, n_pad, h_dim, cpw, with_deg=False)
    (s2,) = seg2(h, adj1)

    out = _tc_layer2(s2, deg3, W2, b2, blk=512)
    return out[:n]


# speedup vs baseline: 1.6835x; 1.6835x over previous
import jax
import jax.numpy as jnp
from jax import lax
from jax.experimental import pallas as pl
from jax.experimental.pallas import tpu as pltpu
from jax.experimental.pallas import tpu_sc as plsc

_NC = 2
_NS = 16
_K = 128
_L = 16
_W = 4


_P0 = 0.29


def _split(e):
    total_chunks = -(-e // _K)
    total_chunks = -(-total_chunks // (2 * _NS)) * (2 * _NS)
    t2 = total_chunks // _NS
    c0 = max(2, round(_P0 * t2))
    c1 = ((t2 - c0) // (2 * _W)) * (2 * _W)
    c0 = t2 - c1
    return c0, c1


def _make_segsum(n_rows, n_pad, f, c0, c1, with_deg):
    stripe = n_pad // _NS
    nw1 = c1 // _W
    mesh = plsc.VectorSubcoreMesh(core_axis_name="core", subcore_axis_name="subcore")
    out_type = [jax.ShapeDtypeStruct((_NC, n_pad, f), jnp.float32)]
    scratch = [
        pltpu.VMEM_SHARED((n_pad, f), jnp.float32),
        pltpu.VMEM((c0, _K), jnp.int32),
        pltpu.VMEM((c0, _K), jnp.int32),
        pltpu.VMEM((2 * _W, _K), jnp.int32),
        pltpu.VMEM((2 * _W, _K), jnp.int32),
        pltpu.VMEM((_K, f), jnp.float32),
        pltpu.VMEM((_K, f), jnp.float32),
        pltpu.SemaphoreType.DMA,
        pltpu.SemaphoreType.DMA,
        pltpu.SemaphoreType.DMA,
        pltpu.SemaphoreType.DMA,
    ]
    if with_deg:
        out_type.append(jax.ShapeDtypeStruct((_NC, n_pad), jnp.float32))
        scratch += [
            pltpu.VMEM((_K,), jnp.float32),
            pltpu.VMEM((stripe,), jnp.float32),
            pltpu.VMEM_SHARED((n_pad,), jnp.float32),
        ]

    def body(vals, src0, dst0, adj1, out, *rest):
        if with_deg:
            deg_out = rest[0]
            rest = rest[1:]
        acc, idx0s, idx0d, ib0, ib1, r0, r1, g0, g1, i0, i1 = rest[:11]
        if with_deg:
            ones, zdeg, accd = rest[11:]
        idxb = (ib0, ib1)
        rows = (r0, r1)
        gsem = (g0, g1)
        isem = (i0, i1)
        c = lax.axis_index("core")
        s = lax.axis_index("subcore")

        zvec = jnp.zeros((_L,), jnp.float32)

        @pl.loop(0, 64)
        def _(i):
            @pl.loop(0, f, step=_L)
            def _(j):
                r0[i, pl.ds(j, _L)] = zvec

        @pl.loop(0, stripe, step=64)
        def _(r):
            pltpu.sync_copy(r0.at[pl.ds(0, 64)],
                            acc.at[pl.ds(s * stripe + r, 64)])

        if with_deg:
            ovec = jnp.ones((_L,), jnp.float32)

            @pl.loop(0, _K, step=_L)
            def _(j):
                ones[pl.ds(j, _L)] = ovec

            @pl.loop(0, stripe, step=_L)
            def _(j):
                zdeg[pl.ds(j, _L)] = zvec

            pltpu.sync_copy(zdeg, accd.at[pl.ds(s * stripe, stripe)])

        plsc.subcore_barrier()

        @pl.when(c == 0)
        def _():
            pltpu.sync_copy(src0.at[s], idx0s)
            pltpu.sync_copy(dst0.at[s], idx0d)

            @pl.loop(0, c0)
            def _(j):
                pltpu.sync_copy(vals.at[idx0s.at[j]], r0)
                pltpu.sync_copy(r0, acc.at[idx0d.at[j]], add=True)
                if with_deg:
                    pltpu.sync_copy(ones, accd.at[idx0d.at[j]], add=True)

        @pl.when(c == 1)
        def _():
            pltpu.sync_copy(adj1.at[s, 0], idxb[0])
            pltpu.async_copy(adj1.at[s, 1], idxb[1], isem[1])
            pltpu.async_copy(vals.at[idxb[0].at[0]], rows[0], gsem[0])
            pltpu.async_copy(vals.at[idxb[0].at[1]], rows[1], gsem[1])

            def half(w, p):
                ib = idxb[p]
                ibn = idxb[1 - p]
                for k in range(_W):
                    b = k % 2
                    pltpu.make_async_copy(
                        vals.at[ib.at[k]], rows[b], gsem[b]).wait()
                    pltpu.sync_copy(rows[b], acc.at[ib.at[_W + k]], add=True)
                    if with_deg:
                        pltpu.sync_copy(ones, accd.at[ib.at[_W + k]],
                                        add=True)
                    if k + 2 < _W:
                        pltpu.async_copy(vals.at[ib.at[k + 2]], rows[b],
                                         gsem[b])
                    elif k == _W - 2:
                        @pl.when(w + 1 < nw1)
                        def _():
                            pltpu.make_async_copy(adj1.at[s, w + 1], ibn,
                                                  isem[1 - p]).wait()
                            pltpu.async_copy(vals.at[ibn.at[0]], rows[b],
                                             gsem[b])
                    else:
                        @pl.when(w + 1 < nw1)
                        def _():
                            pltpu.async_copy(vals.at[ibn.at[1]], rows[b],
                                             gsem[b])

                @pl.when(w + 2 < nw1)
                def _():
                    pltpu.async_copy(adj1.at[s, w + 2], idxb[p], isem[p])

            @pl.loop(0, nw1, step=2)
            def _(w):
                half(w, 0)
                half(w + 1, 1)

        plsc.subcore_barrier()

        pltpu.sync_copy(acc.at[pl.ds(s * stripe, stripe)],
                        out.at[c, pl.ds(s * stripe, stripe)])
        if with_deg:
            pltpu.sync_copy(accd.at[pl.ds(s * stripe, stripe)],
                            deg_out.at[c, pl.ds(s * stripe, stripe)])

    return pl.kernel(body, out_type=tuple(out_type), mesh=mesh,
                     scratch_types=scratch)


def _tc_layer1(s1, deg3, W1, b1, blk):
    n_pad, f_in = s1.shape[1], s1.shape[2]
    h_dim = W1.shape[1]

    def body(p_ref, d_ref, w1_ref, b1_ref, o_ref):
        ssum = p_ref[0] + p_ref[1]
        dsum = d_ref[0] + d_ref[1]
        norm = 1.0 / jnp.maximum(dsum, 1.0)
        agg = jnp.dot(ssum, w1_ref[...], preferred_element_type=jnp.float32)
        agg = (agg + dsum * b1_ref[...]) * norm
        o_ref[...] = jnp.maximum(agg, 0.0)

    return pl.pallas_call(
        body,
        grid=(n_pad // blk,),
        in_specs=[
            pl.BlockSpec((_NC, blk, f_in), lambda i: (0, i, 0)),
            pl.BlockSpec((_NC, blk, 1), lambda i: (0, i, 0)),
            pl.BlockSpec((f_in, h_dim), lambda i: (0, 0)),
            pl.BlockSpec((1, h_dim), lambda i: (0, 0)),
        ],
        out_specs=pl.BlockSpec((blk, h_dim), lambda i: (i, 0)),
        out_shape=jax.ShapeDtypeStruct((n_pad, h_dim), jnp.float32),
    )(s1, deg3, W1, b1.reshape(1, h_dim))


def _tc_layer2(s2, deg3, W2, b2, blk):
    n_pad, h_dim = s2.shape[1], s2.shape[2]
    c_dim = W2.shape[1]

    def body(p_ref, d_ref, w2_ref, b2_ref, o_ref):
        ssum = p_ref[0] + p_ref[1]
        dsum = d_ref[0] + d_ref[1]
        norm = 1.0 / jnp.maximum(dsum, 1.0)
        agg = jnp.dot(ssum, w2_ref[...], preferred_element_type=jnp.float32)
        v = (agg + dsum * b2_ref[...]) * norm
        m = jnp.max(v, axis=1, keepdims=True)
        e = jnp.exp(v - m)
        lse = jnp.log(jnp.sum(e, axis=1, keepdims=True))
        o_ref[...] = (v - m) - lse

    return pl.pallas_call(
        body,
        grid=(n_pad // blk,),
        in_specs=[
            pl.BlockSpec((_NC, blk, h_dim), lambda i: (0, i, 0)),
            pl.BlockSpec((_NC, blk, 1), lambda i: (0, i, 0)),
            pl.BlockSpec((h_dim, c_dim), lambda i: (0, 0)),
            pl.BlockSpec((1, c_dim), lambda i: (0, 0)),
        ],
        out_specs=pl.BlockSpec((blk, c_dim), lambda i: (i, 0)),
        out_shape=jax.ShapeDtypeStruct((n_pad, c_dim), jnp.float32),
    )(s2, deg3, W2, b2.reshape(1, c_dim))


def kernel(x, adj, W1, b1, W2, b2):
    n, f_in = x.shape
    h_dim = W1.shape[1]
    c_dim = W2.shape[1]
    e = adj.shape[1]

    n_pad = ((n + 1023) // 1024) * 1024
    c0, c1 = _split(e)
    e_pad = _NS * (c0 + c1) * _K

    src = adj[0].astype(jnp.int32)
    dst = adj[1].astype(jnp.int32)
    pad_dst = n + jnp.arange(e_pad - e, dtype=jnp.int32) % (n_pad - n)
    srcf = jnp.concatenate([src, jnp.zeros((e_pad - e,), jnp.int32)])
    dstf = jnp.concatenate([dst, pad_dst])

    e0 = _NS * c0 * _K
    src0 = srcf[:e0].reshape(_NS, c0, _K)
    dst0 = dstf[:e0].reshape(_NS, c0, _K)
    adj1 = jnp.concatenate(
        [srcf[e0:].reshape(_NS, c1 // _W, _W, _K),
         dstf[e0:].reshape(_NS, c1 // _W, _W, _K)],
        axis=2)

    seg1 = _make_segsum(n, n_pad, f_in, c0, c1, with_deg=True)
    s1, deg = seg1(x, src0, dst0, adj1)
    deg3 = deg.reshape(_NC, n_pad, 1)

    h = _tc_layer1(s1, deg3, W1, b1, blk=512)

    seg2 = _make_segsum(n_pad, n_pad, h_dim, c0, c1, with_deg=False)
    (s2,) = seg2(h, src0, dst0, adj1)

    out = _tc_layer2(s2, deg3, W2, b2, blk=512)
    return out[:n]
